# baseline (speedup 1.0000x reference)
import functools

import jax
import jax.numpy as jnp
from jax import lax
from jax.experimental import pallas as pl
from jax.experimental.pallas import tpu as pltpu
from jax.experimental.pallas import tpu_sc as plsc


def kernel(input, indices):
    X, Y, N = input.shape
    K = indices.shape[0]
    NC, NS, L = 2, 16, 16
    NW = NC * NS
    SLABS = X // NW
    ROW = Y * N
    OROW = Y * K
    STARTS = [min(c * L, K - L) for c in range(-(-K // L))]

    in_flat = input.reshape(X * ROW)
    mesh = plsc.VectorSubcoreMesh(core_axis_name="c", subcore_axis_name="s")

    @functools.partial(
        pl.kernel,
        mesh=mesh,
        compiler_params=pltpu.CompilerParams(needs_layout_passes=False),
        out_type=jax.ShapeDtypeStruct((X * OROW,), jnp.float32),
        scratch_types=[
            pltpu.VMEM((K,), jnp.int32),
            pltpu.VMEM((ROW,), jnp.float32),
            pltpu.VMEM((OROW,), jnp.float32),
            pltpu.SemaphoreType.DMA,
        ],
    )
    def sc_run(in_hbm, idx_hbm, out_hbm, idx_v, slab_v, oslab_v, sem):
        wid = lax.axis_index("s") * NC + lax.axis_index("c")
        pltpu.sync_copy(idx_hbm, idx_v)

        def slab_body(i, carry):
            slab = wid * SLABS + i
            src = pl.multiple_of(slab * ROW, 8)
            dst = pl.multiple_of(slab * OROW, 8)
            pltpu.async_copy(in_hbm.at[pl.ds(src, ROW)], slab_v, sem).wait()

            def row_body(r, c2):
                base = r * N
                for s in STARTS:
                    idxv = idx_v[pl.ds(s, L)] + base
                    oslab_v[pl.ds(r * K + s, L)] = plsc.load_gather(
                        slab_v, [idxv])
                return c2

            lax.fori_loop(0, Y, row_body, 0)
            pltpu.sync_copy(oslab_v, out_hbm.at[pl.ds(dst, OROW)])
            return carry

        lax.fori_loop(0, SLABS, slab_body, 0)

    out = sc_run(in_flat, indices)
    return out.reshape(X, Y, K)


# speedup vs baseline: 1.3278x; 1.3278x over previous
import functools

import jax
import jax.numpy as jnp
from jax import lax
from jax.experimental import pallas as pl
from jax.experimental.pallas import tpu as pltpu
from jax.experimental.pallas import tpu_sc as plsc


def kernel(input, indices):
    X, Y, N = input.shape
    K = indices.shape[0]
    NC, NS, L = 2, 16, 16
    NW = NC * NS
    SLABS = X // NW
    ROW = Y * N
    OROW = Y * K
    NBUF = 2
    STARTS = [min(c * L, K - L) for c in range(-(-K // L))]

    in_flat = input.reshape(X * ROW)
    mesh = plsc.VectorSubcoreMesh(core_axis_name="c", subcore_axis_name="s")

    @functools.partial(
        pl.kernel,
        mesh=mesh,
        compiler_params=pltpu.CompilerParams(needs_layout_passes=False),
        out_type=jax.ShapeDtypeStruct((X * OROW,), jnp.float32),
        scratch_types=[
            pltpu.VMEM((K,), jnp.int32),
            pltpu.VMEM((NBUF * ROW,), jnp.float32),
            pltpu.VMEM((NBUF * OROW,), jnp.float32),
            [pltpu.SemaphoreType.DMA] * NBUF,
            [pltpu.SemaphoreType.DMA] * NBUF,
        ],
    )
    def sc_run(in_hbm, idx_hbm, out_hbm, idx_v, slab_v, oslab_v,
               in_sems, out_sems):
        wid = lax.axis_index("s") * NC + lax.axis_index("c")
        base0 = wid * SLABS
        pltpu.sync_copy(idx_hbm, idx_v)

        def in_copy(slab, b):
            src = pl.multiple_of(slab * ROW, 8)
            return pltpu.make_async_copy(
                in_hbm.at[pl.ds(src, ROW)],
                slab_v.at[pl.ds(b * ROW, ROW)], in_sems[b])

        def out_copy(slab, b):
            dst = pl.multiple_of(slab * OROW, 8)
            return pltpu.make_async_copy(
                oslab_v.at[pl.ds(b * OROW, OROW)],
                out_hbm.at[pl.ds(dst, OROW)], out_sems[b])

        for b in range(NBUF):
            in_copy(base0 + b, b).start()

        @pl.loop(0, SLABS, step=NBUF)
        def _(g):
            idxc = [idx_v[pl.ds(s, L)] for s in STARTS]
            for b in range(NBUF):
                i = g + b
                slab = base0 + i
                in_copy(slab, b).wait()

                @pl.when(i >= NBUF)
                def _():
                    out_copy(slab, b).wait()

                for r in range(Y):
                    row = slab_v.at[pl.ds(b * ROW + r * N, N)]
                    for ci, s in enumerate(STARTS):
                        oslab_v[pl.ds(b * OROW + r * K + s, L)] = (
                            plsc.load_gather(row, [idxc[ci]]))

                out_copy(slab, b).start()

                @pl.when(i + NBUF < SLABS)
                def _():
                    in_copy(slab + NBUF, b).start()

        for b in range(NBUF):
            out_copy(base0 + SLABS - NBUF + b, b).wait()

    out = sc_run(in_flat, indices)
    return out.reshape(X, Y, K)


# speedup vs baseline: 2.2869x; 1.7223x over previous
import functools

import jax
import jax.numpy as jnp
from jax import lax
from jax.experimental import pallas as pl
from jax.experimental.pallas import tpu as pltpu
from jax.experimental.pallas import tpu_sc as plsc


def kernel(input, indices):
    X, Y, N = input.shape
    K = indices.shape[0]
    NC, NS, L = 2, 16, 16
    NW = NC * NS
    SLABS = X // NW
    NBUF = 2
    STARTS = [min(c * L, K - L) for c in range(-(-K // L))]

    mesh = plsc.VectorSubcoreMesh(core_axis_name="c", subcore_axis_name="s")

    @functools.partial(
        pl.kernel,
        mesh=mesh,
        compiler_params=pltpu.CompilerParams(needs_layout_passes=False),
        out_type=jax.ShapeDtypeStruct((X, Y, K), jnp.float32),
        scratch_types=[
            pltpu.VMEM((K,), jnp.int32),
            pltpu.VMEM((NBUF, Y, N), jnp.float32),
            pltpu.VMEM((NBUF, Y, K), jnp.float32),
            [pltpu.SemaphoreType.DMA] * NBUF,
            [pltpu.SemaphoreType.DMA] * NBUF,
        ],
    )
    def sc_run(in_hbm, idx_hbm, out_hbm, idx_v, slab_v, oslab_v,
               in_sems, out_sems):
        wid = lax.axis_index("s") * NC + lax.axis_index("c")
        base0 = wid * SLABS

        pltpu.sync_copy(idx_hbm, idx_v)

        def in_copy(slab, b):
            return pltpu.make_async_copy(
                in_hbm.at[slab], slab_v.at[b], in_sems[b])

        def out_copy(slab, b):
            return pltpu.make_async_copy(
                oslab_v.at[b], out_hbm.at[slab], out_sems[b])

        for b in range(NBUF):
            in_copy(base0 + b, b).start()

        @pl.loop(0, SLABS, step=NBUF)
        def _(g):
            idxc = [idx_v[pl.ds(s, L)] for s in STARTS]
            for b in range(NBUF):
                i = g + b
                slab = base0 + i
                in_copy(slab, b).wait()

                @pl.when(i >= NBUF)
                def _():
                    out_copy(slab, b).wait()

                for r in range(Y):
                    rvec = jnp.full((L,), r, dtype=jnp.int32)
                    for ci, s in enumerate(STARTS):
                        oslab_v[b, r, pl.ds(s, L)] = plsc.load_gather(
                            slab_v.at[b], [rvec, idxc[ci]])

                out_copy(slab, b).start()

                @pl.when(i + NBUF < SLABS)
                def _():
                    in_copy(slab + NBUF, b).start()

        for b in range(NBUF):
            out_copy(base0 + SLABS - NBUF + b, b).wait()

    return sc_run(input, indices)


# speedup vs baseline: 2.2930x; 1.0026x over previous
import functools

import jax
import jax.numpy as jnp
from jax import lax
from jax.experimental import pallas as pl
from jax.experimental.pallas import tpu as pltpu
from jax.experimental.pallas import tpu_sc as plsc


def kernel(input, indices):
    X, Y, N = input.shape
    K = indices.shape[0]
    NC, NS, L = 2, 16, 16
    NW = NC * NS
    SLABS = X // NW
    NBUF = 2
    STARTS = [min(c * L, K - L) for c in range(-(-K // L))]

    mesh = plsc.VectorSubcoreMesh(core_axis_name="c", subcore_axis_name="s")

    @functools.partial(
        pl.kernel,
        mesh=mesh,
        compiler_params=pltpu.CompilerParams(
            needs_layout_passes=False, use_tc_tiling_on_sc=True),
        out_type=jax.ShapeDtypeStruct((X, Y, K), jnp.float32),
        scratch_types=[
            pltpu.VMEM((K,), jnp.int32),
            pltpu.VMEM((NBUF, Y, N), jnp.float32),
            pltpu.VMEM((NBUF, Y, K), jnp.float32),
            [pltpu.SemaphoreType.DMA] * NBUF,
            [pltpu.SemaphoreType.DMA] * NBUF,
        ],
    )
    def sc_run(in_hbm, idx_hbm, out_hbm, idx_v, slab_v, oslab_v,
               in_sems, out_sems):
        wid = lax.axis_index("s") * NC + lax.axis_index("c")
        base0 = wid * SLABS

        pltpu.sync_copy(idx_hbm, idx_v)

        def in_copy(slab, b):
            return pltpu.make_async_copy(
                in_hbm.at[slab], slab_v.at[b], in_sems[b])

        def out_copy(slab, b):
            return pltpu.make_async_copy(
                oslab_v.at[b], out_hbm.at[slab], out_sems[b])

        for b in range(NBUF):
            in_copy(base0 + b, b).start()

        @pl.loop(0, SLABS, step=NBUF)
        def _(g):
            idxc = [idx_v[pl.ds(s, L)] for s in STARTS]
            for b in range(NBUF):
                i = g + b
                slab = base0 + i
                in_copy(slab, b).wait()

                @pl.when(i >= NBUF)
                def _():
                    out_copy(slab, b).wait()

                for r in range(Y):
                    rvec = jnp.full((L,), r, dtype=jnp.int32)
                    for ci, s in enumerate(STARTS):
                        oslab_v[b, r, pl.ds(s, L)] = plsc.load_gather(
                            slab_v.at[b], [rvec, idxc[ci]])

                out_copy(slab, b).start()

                @pl.when(i + NBUF < SLABS)
                def _():
                    in_copy(slab + NBUF, b).start()

        for b in range(NBUF):
            out_copy(base0 + SLABS - NBUF + b, b).wait()

    return sc_run(input, indices)


# speedup vs baseline: 24.1419x; 10.5287x over previous
import functools

import jax
import jax.numpy as jnp
from jax import lax
from jax.experimental import pallas as pl
from jax.experimental.pallas import tpu as pltpu
from jax.experimental.pallas import tpu_sc as plsc


def kernel(input, indices):
    X, Y, N = input.shape
    K = indices.shape[0]
    NC, NS = 2, 16
    NW = NC * NS
    LT = X // 128
    UPW = (Y * LT) // NW
    NBUF = 4
    HALVES = [(0, 104), (104, 96)]

    t = jnp.transpose(input, (1, 2, 0))

    mesh = plsc.VectorSubcoreMesh(core_axis_name="c", subcore_axis_name="s")

    @functools.partial(
        pl.kernel,
        mesh=mesh,
        compiler_params=pltpu.CompilerParams(
            needs_layout_passes=False, use_tc_tiling_on_sc=True),
        out_type=jax.ShapeDtypeStruct((Y, K, X), jnp.float32),
        scratch_types=[
            pltpu.VMEM((K,), jnp.int32),
            pltpu.VMEM((NBUF, K, 128), jnp.float32),
            pltpu.SemaphoreType.DMA((NBUF,)),
            pltpu.SemaphoreType.DMA((NBUF,)),
        ],
    )
    def sc_run(in_hbm, idx_hbm, out_hbm, idx_v, buf_v, in_sems, out_sems):
        w = lax.axis_index("s") * NC + lax.axis_index("c")
        lane = pl.multiple_of(w * 128, 128)
        pltpu.sync_copy(idx_hbm, idx_v)

        def gather_half(j, b, h):
            o, n = HALVES[h]
            return pltpu.make_async_copy(
                in_hbm.at[j].at[idx_v.at[pl.ds(o, n)], pl.ds(lane, 128)],
                buf_v.at[b, pl.ds(o, n)],
                in_sems.at[b])

        def out_copy(j, b):
            return pltpu.make_async_copy(
                buf_v.at[b],
                out_hbm.at[j, :, pl.ds(lane, 128)],
                out_sems.at[b])

        for u in range(2):
            for h in range(2):
                gather_half(u, u % NBUF, h).start()

        @pl.loop(0, UPW)
        def _(u):
            b = lax.rem(u, NBUF)
            for h in range(2):
                gather_half(u, b, h).wait()
            out_copy(u, b).start()

            @pl.when(u + 2 < UPW)
            def _():
                bn = lax.rem(u + 2, NBUF)

                @pl.when(u >= 2)
                def _():
                    out_copy(u - 2, bn).wait()

                for h in range(2):
                    gather_half(u + 2, bn, h).start()

        for u in range(UPW - NBUF, UPW):
            out_copy(u, u % NBUF).wait()

    out_t = sc_run(t, indices)
    return jnp.transpose(out_t, (2, 0, 1))


# speedup vs baseline: 24.5435x; 1.0166x over previous
import functools

import jax
import jax.numpy as jnp
from jax import lax
from jax.experimental import pallas as pl
from jax.experimental.pallas import tpu as pltpu
from jax.experimental.pallas import tpu_sc as plsc


def kernel(input, indices):
    X, Y, N = input.shape
    K = indices.shape[0]
    NC, NS = 2, 16
    NW = NC * NS
    CC = 8
    NCHUNK = K // CC
    UNITS = Y * NCHUNK
    ITERS = -(-UNITS // NW)
    NBUF = 3

    t = jnp.transpose(input, (1, 2, 0))

    mesh = plsc.VectorSubcoreMesh(core_axis_name="c", subcore_axis_name="s")

    @functools.partial(
        pl.kernel,
        mesh=mesh,
        compiler_params=pltpu.CompilerParams(
            needs_layout_passes=False, use_tc_tiling_on_sc=True),
        out_type=jax.ShapeDtypeStruct((Y, K, X), jnp.float32),
        scratch_types=[
            pltpu.VMEM((K,), jnp.int32),
            pltpu.VMEM((NBUF, CC, X), jnp.float32),
            pltpu.SemaphoreType.DMA((NBUF,)),
            pltpu.SemaphoreType.DMA((NBUF,)),
        ],
    )
    def sc_run(in_hbm, idx_hbm, out_hbm, idx_v, buf_v, in_sems, out_sems):
        w = lax.axis_index("s") * NC + lax.axis_index("c")
        pltpu.sync_copy(idx_hbm, idx_v)

        def unit_jc(i):
            unit = w + i * NW
            return unit // NCHUNK, lax.rem(unit, NCHUNK) * CC

        def gather(i, b):
            j, c0 = unit_jc(i)
            return pltpu.make_async_copy(
                in_hbm.at[j].at[idx_v.at[pl.ds(pl.multiple_of(c0, 8), CC)]],
                buf_v.at[b],
                in_sems.at[b])

        def out_copy(i, b):
            j, c0 = unit_jc(i)
            return pltpu.make_async_copy(
                buf_v.at[b],
                out_hbm.at[j, pl.ds(pl.multiple_of(c0, 8), CC)],
                out_sems.at[b])

        def guarded(i, fn):
            @pl.when(w + i * NW < UNITS)
            def _():
                fn()

        for u in range(2):
            guarded(u, lambda u=u: gather(u, u % NBUF).start())

        @pl.loop(0, ITERS)
        def _(u):
            b = lax.rem(u, NBUF)
            guarded(u, lambda: gather(u, b).wait())
            guarded(u, lambda: out_copy(u, b).start())

            @pl.when(u + 2 < ITERS)
            def _():
                bn = lax.rem(u + 2, NBUF)

                @pl.when(u >= 1)
                def _():
                    guarded(u - 1, lambda: out_copy(u - 1, bn).wait())

                guarded(u + 2, lambda: gather(u + 2, bn).start())

        for u in range(ITERS - NBUF, ITERS):
            guarded(u, lambda u=u: out_copy(u, u % NBUF).wait())

    out_t = sc_run(t, indices)
    return jnp.transpose(out_t, (2, 0, 1))


# speedup vs baseline: 24.9619x; 1.0170x over previous
import functools

import jax
import jax.numpy as jnp
from jax import lax
from jax.experimental import pallas as pl
from jax.experimental.pallas import tpu as pltpu
from jax.experimental.pallas import tpu_sc as plsc


def kernel(input, indices):
    X, Y, N = input.shape
    K = indices.shape[0]
    NC, NS = 2, 16
    NW = NC * NS
    CC = 8
    NCHUNK = K // CC
    XH = X // 2
    SUB = NCHUNK * 2
    UNITS = Y * SUB
    ITERS = -(-UNITS // NW)
    NBUF = 6
    PREF = 4

    t = jnp.transpose(input, (1, 2, 0))

    mesh = plsc.VectorSubcoreMesh(core_axis_name="c", subcore_axis_name="s")

    @functools.partial(
        pl.kernel,
        mesh=mesh,
        compiler_params=pltpu.CompilerParams(
            needs_layout_passes=False, use_tc_tiling_on_sc=True),
        out_type=jax.ShapeDtypeStruct((Y, K, X), jnp.float32),
        scratch_types=[
            pltpu.VMEM((K,), jnp.int32),
            pltpu.VMEM((NBUF, CC, XH), jnp.float32),
            pltpu.SemaphoreType.DMA((NBUF,)),
            pltpu.SemaphoreType.DMA((NBUF,)),
        ],
    )
    def sc_run(in_hbm, idx_hbm, out_hbm, idx_v, buf_v, in_sems, out_sems):
        w = lax.axis_index("s") * NC + lax.axis_index("c")
        pltpu.sync_copy(idx_hbm, idx_v)

        def unit_jch(i):
            unit = w + i * NW
            sub = lax.rem(unit, SUB)
            return (unit // SUB,
                    pl.multiple_of((sub // 2) * CC, CC),
                    pl.multiple_of(lax.rem(sub, 2) * XH, 128))

        def gather(i, b):
            j, c0, l0 = unit_jch(i)
            return pltpu.make_async_copy(
                in_hbm.at[j].at[idx_v.at[pl.ds(c0, CC)], pl.ds(l0, XH)],
                buf_v.at[b],
                in_sems.at[b])

        def out_copy(i, b):
            j, c0, l0 = unit_jch(i)
            return pltpu.make_async_copy(
                buf_v.at[b],
                out_hbm.at[j, pl.ds(c0, CC), pl.ds(l0, XH)],
                out_sems.at[b])

        def guarded(i, fn):
            @pl.when(w + i * NW < UNITS)
            def _():
                fn()

        for u in range(PREF):
            guarded(u, lambda u=u: gather(u, u % NBUF).start())

        @pl.loop(0, ITERS)
        def _(u):
            b = lax.rem(u, NBUF)
            guarded(u, lambda: gather(u, b).wait())
            guarded(u, lambda: out_copy(u, b).start())

            @pl.when(u + PREF < ITERS)
            def _():
                bn = lax.rem(u + PREF, NBUF)

                @pl.when(u >= NBUF - PREF)
                def _():
                    guarded(u - (NBUF - PREF),
                            lambda: out_copy(u - (NBUF - PREF), bn).wait())

                guarded(u + PREF, lambda: gather(u + PREF, bn).start())

        for u in range(ITERS - NBUF, ITERS):
            guarded(u, lambda u=u: out_copy(u, u % NBUF).wait())

    out_t = sc_run(t, indices)
    return jnp.transpose(out_t, (2, 0, 1))
